# baseline (speedup 1.0000x reference)
import functools

import jax
import jax.numpy as jnp
from jax import lax
from jax.experimental import pallas as pl
from jax.experimental.pallas import tpu as pltpu
from jax.experimental.pallas import tpu_sc as plsc

B, C, H, W = 32, 384, 48, 48
R = B * H * W
NC, NS = 2, 16
NW = NC * NS
ROWS = R // NW
RB = 64
NB = ROWS // RB
DEPTH = 4

_mesh = plsc.VectorSubcoreMesh(core_axis_name="c", subcore_axis_name="s")


@functools.partial(
    pl.kernel,
    mesh=_mesh,
    out_type=jax.ShapeDtypeStruct((R, C), jnp.float32),
    scratch_types=(
        [pltpu.VMEM((RB, C), jnp.float32)] * DEPTH
        + [pltpu.SemaphoreType.DMA] * (2 * DEPTH)
    ),
)
def _copy_kernel(x_hbm, out_hbm, *scratch):
    bufs = list(scratch[:DEPTH])
    sin = list(scratch[DEPTH:2 * DEPTH])
    sout = list(scratch[2 * DEPTH:])
    wid = lax.axis_index("s") * NC + lax.axis_index("c")
    base = pl.multiple_of(wid * ROWS, 8)
    loads = [None] * DEPTH
    stores = [None] * DEPTH

    def start_load(i):
        b = i % DEPTH
        if stores[b] is not None:
            stores[b].wait()
        loads[b] = pltpu.make_async_copy(
            x_hbm.at[pl.ds(base + i * RB, RB)], bufs[b], sin[b])
        loads[b].start()

    for i in range(DEPTH - 1):
        start_load(i)
    for i in range(NB):
        b = i % DEPTH
        if i + DEPTH - 1 < NB:
            start_load(i + DEPTH - 1)
        loads[b].wait()
        stores[b] = pltpu.make_async_copy(
            bufs[b], out_hbm.at[pl.ds(base + i * RB, RB)], sout[b])
        stores[b].start()
    for b in range(DEPTH):
        if stores[b] is not None:
            stores[b].wait()


def kernel(x):
    xt = x.transpose(0, 2, 3, 1).reshape(R, C)
    out = _copy_kernel(xt)
    return out.reshape(B, H, W, C).transpose(0, 3, 1, 2)


# speedup vs baseline: 1.0190x; 1.0190x over previous
import functools

import jax
import jax.numpy as jnp
from jax import lax
from jax.experimental import pallas as pl
from jax.experimental.pallas import tpu as pltpu
from jax.experimental.pallas import tpu_sc as plsc

B, C, H, W = 32, 384, 48, 48
R = B * H * W
NC, NS = 2, 16
NW = NC * NS
ROWS = R // NW
RB = 144
NB = ROWS // RB
DEPTH = 2

_mesh = plsc.VectorSubcoreMesh(core_axis_name="c", subcore_axis_name="s")


@functools.partial(
    pl.kernel,
    mesh=_mesh,
    out_type=jax.ShapeDtypeStruct((R, C), jnp.float32),
    scratch_types=(
        [pltpu.VMEM((RB, C), jnp.float32)] * DEPTH
        + [pltpu.SemaphoreType.DMA] * (2 * DEPTH)
    ),
)
def _copy_kernel(x_hbm, out_hbm, *scratch):
    bufs = list(scratch[:DEPTH])
    sin = list(scratch[DEPTH:2 * DEPTH])
    sout = list(scratch[2 * DEPTH:])
    wid = lax.axis_index("s") * NC + lax.axis_index("c")
    base = pl.multiple_of(wid * ROWS, 8)
    loads = [None] * DEPTH
    stores = [None] * DEPTH

    def start_load(i):
        b = i % DEPTH
        if stores[b] is not None:
            stores[b].wait()
        loads[b] = pltpu.make_async_copy(
            x_hbm.at[pl.ds(base + i * RB, RB)], bufs[b], sin[b])
        loads[b].start()

    for i in range(DEPTH - 1):
        start_load(i)
    for i in range(NB):
        b = i % DEPTH
        if i + DEPTH - 1 < NB:
            start_load(i + DEPTH - 1)
        loads[b].wait()
        stores[b] = pltpu.make_async_copy(
            bufs[b], out_hbm.at[pl.ds(base + i * RB, RB)], sout[b])
        stores[b].start()
    for b in range(DEPTH):
        if stores[b] is not None:
            stores[b].wait()


def kernel(x):
    xt = x.transpose(0, 2, 3, 1).reshape(R, C)
    out = _copy_kernel(xt)
    return out.reshape(B, H, W, C).transpose(0, 3, 1, 2)
